# baseline (speedup 1.0000x reference)
import functools

import jax
import jax.numpy as jnp
from jax import lax
from jax.experimental import pallas as pl
from jax.experimental.pallas import tpu as pltpu
from jax.experimental.pallas import tpu_sc as plsc

B = 16384
D = 32
IDS_W = 128
NC, NS = 2, 16
NW = NC * NS
ROWS_PER_W = B // NW
CHUNKS = ROWS_PER_W // IDS_W


def _sc_gather_body(ids_hbm, ut_hbm, it_hbm, ct_hbm,
                    uout_hbm, iout_hbm, cout_hbm,
                    uidx, iidx, cidx, urows, irows, crows, sem):
    wid = lax.axis_index("s") * NC + lax.axis_index("c")
    r0 = wid * CHUNKS
    base = wid * ROWS_PER_W

    pltpu.sync_copy(ids_hbm.at[0, pl.ds(r0, CHUNKS)], uidx)
    pltpu.sync_copy(ids_hbm.at[1, pl.ds(r0, CHUNKS)], iidx)
    pltpu.sync_copy(ids_hbm.at[2, pl.ds(r0, CHUNKS)], cidx)

    descs = []
    for j in range(CHUNKS):
        dst = pl.ds(j * IDS_W, IDS_W)
        descs.append(pltpu.async_copy(ut_hbm.at[uidx.at[j]], urows.at[dst], sem))
        descs.append(pltpu.async_copy(it_hbm.at[iidx.at[j]], irows.at[dst], sem))
        descs.append(pltpu.async_copy(ct_hbm.at[cidx.at[j]], crows.at[dst], sem))
    for dsc in descs:
        dsc.wait()

    rows = pl.ds(base, ROWS_PER_W)
    pltpu.sync_copy(urows, uout_hbm.at[rows])
    pltpu.sync_copy(irows, iout_hbm.at[rows])
    pltpu.sync_copy(crows, cout_hbm.at[rows])


@functools.cache
def _sc_gather_kernel():
    return pl.kernel(
        _sc_gather_body,
        out_type=(
            jax.ShapeDtypeStruct((B, D), jnp.float32),
            jax.ShapeDtypeStruct((B, D), jnp.float32),
            jax.ShapeDtypeStruct((B, D), jnp.float32),
        ),
        mesh=plsc.VectorSubcoreMesh(core_axis_name="c", subcore_axis_name="s"),
        compiler_params=pltpu.CompilerParams(use_tc_tiling_on_sc=False),
        scratch_types=[
            pltpu.VMEM((CHUNKS, IDS_W), jnp.int32),
            pltpu.VMEM((CHUNKS, IDS_W), jnp.int32),
            pltpu.VMEM((CHUNKS, IDS_W), jnp.int32),
            pltpu.VMEM((ROWS_PER_W, D), jnp.float32),
            pltpu.VMEM((ROWS_PER_W, D), jnp.float32),
            pltpu.VMEM((ROWS_PER_W, D), jnp.float32),
            pltpu.SemaphoreType.DMA,
        ],
    )


_MLP_BLK = 2048


def _mlp_body(u_ref, i_ref, c_ref, dense_ref, w1u_ref, w1i_ref, w1c_ref,
              w1d_ref, b1_ref, w2_ref, b2_ref, out_ref):
    d = dense_ref[...]
    h = jnp.dot(u_ref[...], w1u_ref[...], preferred_element_type=jnp.float32)
    h = h + jnp.dot(i_ref[...], w1i_ref[...], preferred_element_type=jnp.float32)
    h = h + jnp.dot(c_ref[...], w1c_ref[...], preferred_element_type=jnp.float32)
    h = h + d[:, 0:1] * w1d_ref[0:1, :] + d[:, 1:2] * w1d_ref[1:2, :]
    h = jnp.maximum(h + b1_ref[...], 0.0)
    o = jnp.sum(h * w2_ref[...], axis=1, keepdims=True) + b2_ref[...]
    out_ref[...] = 1.0 / (1.0 + jnp.exp(-o))


def _tc_mlp(u, i, c, dense, w1u, w1i, w1c, w1d, b1r, w2r, b2r):
    grid = (B // _MLP_BLK,)
    feat_spec = pl.BlockSpec((_MLP_BLK, D), lambda i: (i, 0))
    w_spec = pl.BlockSpec((D, 16), lambda i: (0, 0))
    return pl.pallas_call(
        _mlp_body,
        grid=grid,
        in_specs=[
            feat_spec, feat_spec, feat_spec,
            pl.BlockSpec((_MLP_BLK, 2), lambda i: (i, 0)),
            w_spec, w_spec, w_spec,
            pl.BlockSpec((2, 16), lambda i: (0, 0)),
            pl.BlockSpec((1, 16), lambda i: (0, 0)),
            pl.BlockSpec((1, 16), lambda i: (0, 0)),
            pl.BlockSpec((1, 1), lambda i: (0, 0)),
        ],
        out_specs=pl.BlockSpec((_MLP_BLK, 1), lambda i: (i, 0)),
        out_shape=jax.ShapeDtypeStruct((B, 1), jnp.float32),
    )(u, i, c, dense, w1u, w1i, w1c, w1d, b1r, w2r, b2r)


def kernel(user_id, item_id, category_id, dense, user_table, item_table,
           cat_table, W1, b1, W2, b2):
    ids = jnp.stack([
        user_id.astype(jnp.int32),
        item_id.astype(jnp.int32),
        category_id.astype(jnp.int32),
    ]).reshape(3, B // IDS_W, IDS_W)
    u, i, c = _sc_gather_kernel()(ids, user_table, item_table, cat_table)
    return _tc_mlp(
        u, i, c, dense,
        W1[:D], W1[D:2 * D], W1[2 * D:3 * D], W1[3 * D:],
        b1.reshape(1, 16), W2.reshape(1, 16), b2.reshape(1, 1),
    )


# speedup vs baseline: 2.9838x; 2.9838x over previous
import functools

import jax
import jax.numpy as jnp
from jax import lax
from jax.experimental import pallas as pl
from jax.experimental.pallas import tpu as pltpu
from jax.experimental.pallas import tpu_sc as plsc

B = 16384
D = 32
NC, NS = 2, 16
NW = NC * NS
ROWS_PER_W = B // NW
CHUNK = 4
NCHUNK = ROWS_PER_W // CHUNK
IDS_W = 128
CAT_CHUNKS = ROWS_PER_W // IDS_W


def _tile_gather_body(ids_hbm, ut_hbm, it_hbm, uout_hbm, iout_hbm,
                      ids_v, buf0, buf1, stage, sem0, sem1):
    wid = lax.axis_index("s") * NC + lax.axis_index("c")
    base = wid * ROWS_PER_W
    lane = lax.iota(jnp.int32, 16)
    zeros16 = lane * 0.0

    @pl.loop(0, ROWS_PER_W)
    def _zero(k):
        for cc in range(2, 8):
            stage[k, pl.ds(cc * 16, 16)] = zeros16

    for t, (src, outref) in enumerate(((ut_hbm, uout_hbm), (it_hbm, iout_hbm))):
        pltpu.sync_copy(ids_hbm.at[t, pl.ds(base, ROWS_PER_W)],
                        ids_v.at[pl.ds(0, ROWS_PER_W)])

        bufs = (buf0, buf1)
        sems = (sem0, sem1)

        def fire(c, slot):
            v = ids_v[pl.ds(c * CHUNK, 16)]
            for kk in range(CHUNK):
                col0 = pl.multiple_of((v[kk] >> 7) * 128, 128)
                pltpu.async_copy(
                    src.at[:, pl.ds(col0, 128)], bufs[slot].at[kk], sems[slot])

        def drain(slot):
            for kk in range(CHUNK):
                pltpu.make_async_copy(
                    src.at[:, pl.ds(0, 128)], bufs[slot].at[kk],
                    sems[slot]).wait()

        def extract(c, slot):
            v = ids_v[pl.ds(c * CHUNK, 16)]
            for kk in range(CHUNK):
                j = lane * 0 + (v[kk] & 127)
                k = c * CHUNK + kk
                kk_b = lane * 0 + kk
                stage[k, pl.ds(0, 16)] = plsc.load_gather(
                    bufs[slot], [kk_b, lane, j])
                stage[k, pl.ds(16, 16)] = plsc.load_gather(
                    bufs[slot], [kk_b, lane + 16, j])

        fire(0, 0)

        @pl.loop(0, NCHUNK, step=2)
        def _round(g):
            fire(g + 1, 1)
            drain(0)
            extract(g, 0)

            @pl.when(g + 2 < NCHUNK)
            def _():
                fire(g + 2, 0)

            drain(1)
            extract(g + 1, 1)

        pltpu.sync_copy(stage, outref.at[pl.ds(base, ROWS_PER_W)])


@functools.cache
def _tile_gather_kernel():
    return pl.kernel(
        _tile_gather_body,
        out_type=(
            jax.ShapeDtypeStruct((B, 128), jnp.float32),
            jax.ShapeDtypeStruct((B, 128), jnp.float32),
        ),
        mesh=plsc.VectorSubcoreMesh(core_axis_name="c", subcore_axis_name="s"),
        compiler_params=pltpu.CompilerParams(needs_layout_passes=False),
        scratch_types=[
            pltpu.VMEM((ROWS_PER_W + 16,), jnp.int32),
            pltpu.VMEM((CHUNK, D, 128), jnp.float32),
            pltpu.VMEM((CHUNK, D, 128), jnp.float32),
            pltpu.VMEM((ROWS_PER_W, 128), jnp.float32),
            pltpu.SemaphoreType.DMA,
            pltpu.SemaphoreType.DMA,
        ],
    )


def _cat_gather_body(ids_hbm, ct_hbm, cout_hbm, cidx, crows, sem):
    wid = lax.axis_index("s") * NC + lax.axis_index("c")
    r0 = wid * CAT_CHUNKS
    base = wid * ROWS_PER_W

    pltpu.sync_copy(ids_hbm.at[pl.ds(r0, CAT_CHUNKS)], cidx)
    descs = []
    for j in range(CAT_CHUNKS):
        dst = pl.ds(j * IDS_W, IDS_W)
        descs.append(pltpu.async_copy(ct_hbm.at[cidx.at[j]], crows.at[dst], sem))
    for dsc in descs:
        dsc.wait()
    pltpu.sync_copy(crows, cout_hbm.at[pl.ds(base, ROWS_PER_W)])


@functools.cache
def _cat_gather_kernel():
    return pl.kernel(
        _cat_gather_body,
        out_type=jax.ShapeDtypeStruct((B, D), jnp.float32),
        mesh=plsc.VectorSubcoreMesh(core_axis_name="c", subcore_axis_name="s"),
        compiler_params=pltpu.CompilerParams(use_tc_tiling_on_sc=False),
        scratch_types=[
            pltpu.VMEM((CAT_CHUNKS, IDS_W), jnp.int32),
            pltpu.VMEM((ROWS_PER_W, D), jnp.float32),
            pltpu.SemaphoreType.DMA,
        ],
    )


_MLP_BLK = 2048


def _mlp_body(u_ref, i_ref, c_ref, dense_ref, w1u_ref, w1i_ref, w1c_ref,
              w1d_ref, b1_ref, w2_ref, b2_ref, out_ref):
    d = dense_ref[...]
    h = jnp.dot(u_ref[...], w1u_ref[...], preferred_element_type=jnp.float32)
    h = h + jnp.dot(i_ref[...], w1i_ref[...], preferred_element_type=jnp.float32)
    h = h + jnp.dot(c_ref[...], w1c_ref[...], preferred_element_type=jnp.float32)
    h = h + d[:, 0:1] * w1d_ref[0:1, :] + d[:, 1:2] * w1d_ref[1:2, :]
    h = jnp.maximum(h + b1_ref[...], 0.0)
    o = jnp.sum(h * w2_ref[...], axis=1, keepdims=True) + b2_ref[...]
    out_ref[...] = 1.0 / (1.0 + jnp.exp(-o))


def _tc_mlp(u, i, c, dense, w1u, w1i, w1c, w1d, b1r, w2r, b2r):
    grid = (B // _MLP_BLK,)
    wide_spec = pl.BlockSpec((_MLP_BLK, 128), lambda i: (i, 0))
    return pl.pallas_call(
        _mlp_body,
        grid=grid,
        in_specs=[
            wide_spec, wide_spec,
            pl.BlockSpec((_MLP_BLK, D), lambda i: (i, 0)),
            pl.BlockSpec((_MLP_BLK, 2), lambda i: (i, 0)),
            pl.BlockSpec((128, 16), lambda i: (0, 0)),
            pl.BlockSpec((128, 16), lambda i: (0, 0)),
            pl.BlockSpec((D, 16), lambda i: (0, 0)),
            pl.BlockSpec((2, 16), lambda i: (0, 0)),
            pl.BlockSpec((1, 16), lambda i: (0, 0)),
            pl.BlockSpec((1, 16), lambda i: (0, 0)),
            pl.BlockSpec((1, 1), lambda i: (0, 0)),
        ],
        out_specs=pl.BlockSpec((_MLP_BLK, 1), lambda i: (i, 0)),
        out_shape=jax.ShapeDtypeStruct((B, 1), jnp.float32),
    )(u, i, c, dense, w1u, w1i, w1c, w1d, b1r, w2r, b2r)


def kernel(user_id, item_id, category_id, dense, user_table, item_table,
           cat_table, W1, b1, W2, b2):
    ids2 = jnp.stack([user_id.astype(jnp.int32), item_id.astype(jnp.int32)])
    u, i = _tile_gather_kernel()(ids2, user_table.T, item_table.T)
    c = _cat_gather_kernel()(
        category_id.astype(jnp.int32).reshape(B // IDS_W, IDS_W), cat_table)
    pad = jnp.zeros((128 - D, 16), jnp.float32)
    return _tc_mlp(
        u, i, c, dense,
        jnp.concatenate([W1[:D], pad]),
        jnp.concatenate([W1[D:2 * D], pad]),
        W1[2 * D:3 * D], W1[3 * D:],
        b1.reshape(1, 16), W2.reshape(1, 16), b2.reshape(1, 1),
    )


# speedup vs baseline: 3.4975x; 1.1722x over previous
import functools

import jax
import jax.numpy as jnp
from jax import lax
from jax.experimental import pallas as pl
from jax.experimental.pallas import tpu as pltpu
from jax.experimental.pallas import tpu_sc as plsc

B = 16384
D = 32
NC, NS = 2, 16
NW = NC * NS
ROWS_PER_W = B // NW
CHUNK = 4
NCHUNK = ROWS_PER_W // CHUNK
IDS_W = 128
CAT_CHUNKS = ROWS_PER_W // IDS_W


HALF = ROWS_PER_W // 2
NBUF = 4
HCHUNK = HALF // CHUNK


def _tile_gather_body(ids_hbm, ut_hbm, it_hbm, uout_hbm, iout_hbm,
                      ids_v, buf0, buf1, buf2, buf3, stage,
                      sem0, sem1, sem2, sem3):
    wid = lax.axis_index("s") * NC + lax.axis_index("c")
    base = wid * ROWS_PER_W
    lane = lax.iota(jnp.int32, 16)
    zeros16 = lane * 0.0

    @pl.loop(0, HALF)
    def _zero(k):
        for cc in range(2, 8):
            stage[k, pl.ds(cc * 16, 16)] = zeros16

    bufs = (buf0, buf1, buf2, buf3)
    sems = (sem0, sem1, sem2, sem3)

    for t, (src, outref) in enumerate(((ut_hbm, uout_hbm), (it_hbm, iout_hbm))):
        pltpu.sync_copy(ids_hbm.at[t, pl.ds(base, ROWS_PER_W)],
                        ids_v.at[pl.ds(0, ROWS_PER_W)])

        def fire(c, slot):
            v = ids_v[pl.ds(c * CHUNK, 16)]
            for kk in range(CHUNK):
                col0 = pl.multiple_of((v[kk] >> 7) * 128, 128)
                pltpu.async_copy(
                    src.at[:, pl.ds(col0, 128)], bufs[slot].at[kk], sems[slot])

        def drain(slot):
            for kk in range(CHUNK):
                pltpu.make_async_copy(
                    src.at[:, pl.ds(0, 128)], bufs[slot].at[kk],
                    sems[slot]).wait()

        def extract(c, k0, slot):
            v = ids_v[pl.ds(c * CHUNK, 16)]
            for kk in range(CHUNK):
                j = lane * 0 + (v[kk] & 127)
                k = k0 + kk
                kk_b = lane * 0 + kk
                stage[k, pl.ds(0, 16)] = plsc.load_gather(
                    bufs[slot], [kk_b, lane, j])
                stage[k, pl.ds(16, 16)] = plsc.load_gather(
                    bufs[slot], [kk_b, lane + 16, j])

        for h in range(2):
            c0 = h * HCHUNK
            for u in range(NBUF):
                fire(c0 + u, u)

            @pl.loop(0, HCHUNK, step=NBUF)
            def _round(g):
                for u in range(NBUF):
                    drain(u)
                    extract(c0 + g + u, (g + u) * CHUNK, u)
                    nxt = g + u + NBUF

                    @pl.when(nxt < HCHUNK)
                    def _():
                        fire(c0 + nxt, u)

            pltpu.sync_copy(stage, outref.at[pl.ds(base + h * HALF, HALF)])


@functools.cache
def _tile_gather_kernel():
    return pl.kernel(
        _tile_gather_body,
        out_type=(
            jax.ShapeDtypeStruct((B, 128), jnp.float32),
            jax.ShapeDtypeStruct((B, 128), jnp.float32),
        ),
        mesh=plsc.VectorSubcoreMesh(core_axis_name="c", subcore_axis_name="s"),
        compiler_params=pltpu.CompilerParams(needs_layout_passes=False),
        scratch_types=[
            pltpu.VMEM((ROWS_PER_W + 16,), jnp.int32),
            pltpu.VMEM((CHUNK, D, 128), jnp.float32),
            pltpu.VMEM((CHUNK, D, 128), jnp.float32),
            pltpu.VMEM((CHUNK, D, 128), jnp.float32),
            pltpu.VMEM((CHUNK, D, 128), jnp.float32),
            pltpu.VMEM((HALF, 128), jnp.float32),
            pltpu.SemaphoreType.DMA,
            pltpu.SemaphoreType.DMA,
            pltpu.SemaphoreType.DMA,
            pltpu.SemaphoreType.DMA,
        ],
    )


def _cat_gather_body(ids_hbm, ct_hbm, cout_hbm, cidx, crows, sem):
    wid = lax.axis_index("s") * NC + lax.axis_index("c")
    r0 = wid * CAT_CHUNKS
    base = wid * ROWS_PER_W

    pltpu.sync_copy(ids_hbm.at[pl.ds(r0, CAT_CHUNKS)], cidx)
    descs = []
    for j in range(CAT_CHUNKS):
        dst = pl.ds(j * IDS_W, IDS_W)
        descs.append(pltpu.async_copy(ct_hbm.at[cidx.at[j]], crows.at[dst], sem))
    for dsc in descs:
        dsc.wait()
    pltpu.sync_copy(crows, cout_hbm.at[pl.ds(base, ROWS_PER_W)])


@functools.cache
def _cat_gather_kernel():
    return pl.kernel(
        _cat_gather_body,
        out_type=jax.ShapeDtypeStruct((B, D), jnp.float32),
        mesh=plsc.VectorSubcoreMesh(core_axis_name="c", subcore_axis_name="s"),
        compiler_params=pltpu.CompilerParams(use_tc_tiling_on_sc=False),
        scratch_types=[
            pltpu.VMEM((CAT_CHUNKS, IDS_W), jnp.int32),
            pltpu.VMEM((ROWS_PER_W, D), jnp.float32),
            pltpu.SemaphoreType.DMA,
        ],
    )


_MLP_BLK = 2048


def _mlp_body(u_ref, i_ref, c_ref, dense_ref, w1u_ref, w1i_ref, w1c_ref,
              w1d_ref, b1_ref, w2_ref, b2_ref, out_ref):
    d = dense_ref[...]
    h = jnp.dot(u_ref[...], w1u_ref[...], preferred_element_type=jnp.float32)
    h = h + jnp.dot(i_ref[...], w1i_ref[...], preferred_element_type=jnp.float32)
    h = h + jnp.dot(c_ref[...], w1c_ref[...], preferred_element_type=jnp.float32)
    h = h + d[:, 0:1] * w1d_ref[0:1, :] + d[:, 1:2] * w1d_ref[1:2, :]
    h = jnp.maximum(h + b1_ref[...], 0.0)
    o = jnp.sum(h * w2_ref[...], axis=1, keepdims=True) + b2_ref[...]
    out_ref[...] = 1.0 / (1.0 + jnp.exp(-o))


def _tc_mlp(u, i, c, dense, w1u, w1i, w1c, w1d, b1r, w2r, b2r):
    grid = (B // _MLP_BLK,)
    wide_spec = pl.BlockSpec((_MLP_BLK, 128), lambda i: (i, 0))
    return pl.pallas_call(
        _mlp_body,
        grid=grid,
        in_specs=[
            wide_spec, wide_spec,
            pl.BlockSpec((_MLP_BLK, D), lambda i: (i, 0)),
            pl.BlockSpec((_MLP_BLK, 2), lambda i: (i, 0)),
            pl.BlockSpec((128, 16), lambda i: (0, 0)),
            pl.BlockSpec((128, 16), lambda i: (0, 0)),
            pl.BlockSpec((D, 16), lambda i: (0, 0)),
            pl.BlockSpec((2, 16), lambda i: (0, 0)),
            pl.BlockSpec((1, 16), lambda i: (0, 0)),
            pl.BlockSpec((1, 16), lambda i: (0, 0)),
            pl.BlockSpec((1, 1), lambda i: (0, 0)),
        ],
        out_specs=pl.BlockSpec((_MLP_BLK, 1), lambda i: (i, 0)),
        out_shape=jax.ShapeDtypeStruct((B, 1), jnp.float32),
    )(u, i, c, dense, w1u, w1i, w1c, w1d, b1r, w2r, b2r)


def kernel(user_id, item_id, category_id, dense, user_table, item_table,
           cat_table, W1, b1, W2, b2):
    ids2 = jnp.stack([user_id.astype(jnp.int32), item_id.astype(jnp.int32)])
    u, i = _tile_gather_kernel()(ids2, user_table.T, item_table.T)
    c = _cat_gather_kernel()(
        category_id.astype(jnp.int32).reshape(B // IDS_W, IDS_W), cat_table)
    pad = jnp.zeros((128 - D, 16), jnp.float32)
    return _tc_mlp(
        u, i, c, dense,
        jnp.concatenate([W1[:D], pad]),
        jnp.concatenate([W1[D:2 * D], pad]),
        W1[2 * D:3 * D], W1[3 * D:],
        b1.reshape(1, 16), W2.reshape(1, 16), b2.reshape(1, 1),
    )
